# baseline (speedup 1.0000x reference)
import jax
import jax.numpy as jnp
from jax.experimental import pallas as pl
from jax.experimental.pallas import tpu as pltpu

_NEG = 0.5
_BE = 512


def _leaky(x):
    return jnp.where(x >= 0, x, _NEG * x)


def _body(a_ref, e_ref, o_ref, e_sc):
    k = pl.program_id(0)
    ne = pl.num_programs(0)

    @pl.when(k == 0)
    def _():
        e_sc[...] = e_ref[...].astype(jnp.bfloat16)
        o_ref[...] = jnp.zeros_like(o_ref)

    ab = a_ref[...].astype(jnp.bfloat16)
    hk = _leaky(jax.lax.dot_general(
        ab, e_sc[...], (((0,), (0,)), ((), ())),
        preferred_element_type=jnp.float32)).astype(jnp.bfloat16)
    o_ref[...] += jax.lax.dot_general(
        ab, hk, (((1,), (0,)), ((), ())),
        preferred_element_type=jnp.float32)

    @pl.when(k == ne - 1)
    def _():
        o_ref[...] = _leaky(o_ref[...])


def kernel(adj, embeds):
    n, e = adj.shape
    d = embeds.shape[1]
    ne = e // _BE
    return pl.pallas_call(
        _body,
        grid=(ne,),
        in_specs=[
            pl.BlockSpec((n, _BE), lambda k: (0, k)),
            pl.BlockSpec((n, d), lambda k: (0, 0)),
        ],
        out_specs=pl.BlockSpec((n, d), lambda k: (0, 0)),
        out_shape=jax.ShapeDtypeStruct((n, d), jnp.float32),
        scratch_shapes=[
            pltpu.VMEM((n, d), jnp.bfloat16),
        ],
        compiler_params=pltpu.CompilerParams(
            vmem_limit_bytes=63 * 1024 * 1024),
    )(adj, embeds)


# speedup vs baseline: 1.4849x; 1.4849x over previous
import jax
import jax.numpy as jnp
from jax.experimental import pallas as pl
from jax.experimental.pallas import tpu as pltpu

_NEG = 0.5
_BE = 256


def _leaky(x):
    return jnp.where(x >= 0, x, _NEG * x)


def _body(a_ref, e_ref, o_ref, e_sc):
    k = pl.program_id(0)
    ne = pl.num_programs(0)

    @pl.when(k == 0)
    def _():
        e_sc[...] = e_ref[...].astype(jnp.bfloat16)
        o_ref[...] = jnp.zeros_like(o_ref)

    n = a_ref.shape[0]
    nc = 4
    h = n // nc
    abs_ = [a_ref[pl.ds(c * h, h), :].astype(jnp.bfloat16) for c in range(nc)]
    parts = [jax.lax.dot_general(
        abs_[c], e_sc[pl.ds(c * h, h), :], (((0,), (0,)), ((), ())),
        preferred_element_type=jnp.float32) for c in range(nc)]
    hk = _leaky(sum(parts)).astype(jnp.bfloat16)
    for c in range(nc):
        o_ref[pl.ds(c * h, h), :] += jax.lax.dot_general(
            abs_[c], hk, (((1,), (0,)), ((), ())),
            preferred_element_type=jnp.float32)

    @pl.when(k == ne - 1)
    def _():
        o_ref[...] = _leaky(o_ref[...])


def kernel(adj, embeds):
    n, e = adj.shape
    d = embeds.shape[1]
    ne = e // _BE
    return pl.pallas_call(
        _body,
        grid=(ne,),
        in_specs=[
            pl.BlockSpec((n, _BE), lambda k: (0, k)),
            pl.BlockSpec((n, d), lambda k: (0, 0)),
        ],
        out_specs=pl.BlockSpec((n, d), lambda k: (0, 0)),
        out_shape=jax.ShapeDtypeStruct((n, d), jnp.float32),
        scratch_shapes=[
            pltpu.VMEM((n, d), jnp.bfloat16),
        ],
        compiler_params=pltpu.CompilerParams(
            vmem_limit_bytes=63 * 1024 * 1024),
    )(adj, embeds)
